# baseline (speedup 1.0000x reference)
import jax
import jax.numpy as jnp
from jax.experimental import pallas as pl


def _pos_kernel(row_ref, col_ref, out_ref):
    H, W, D = 32, 32, 256
    col = col_ref[0:W, :]
    row = row_ref[0:H, :]
    colT = col.T
    rowT = row.T
    x_part = jnp.broadcast_to(colT[:, None, :], (D, H, W))
    y_part = jnp.broadcast_to(rowT[:, :, None], (D, H, W))
    out_ref[0] = jnp.concatenate([x_part, y_part], axis=0)


def kernel(row_embeddings, column_embeddings, x):
    batch, _, height, width = x.shape
    D = row_embeddings.shape[1]
    out = pl.pallas_call(
        _pos_kernel,
        grid=(batch,),
        in_specs=[
            pl.BlockSpec(row_embeddings.shape, lambda b: (0, 0)),
            pl.BlockSpec(column_embeddings.shape, lambda b: (0, 0)),
        ],
        out_specs=pl.BlockSpec((1, 2 * D, height, width), lambda b: (b, 0, 0, 0)),
        out_shape=jax.ShapeDtypeStruct((batch, 2 * D, height, width), jnp.float32),
    )(row_embeddings, column_embeddings)
    return out
